# Initial kernel scaffold; baseline (speedup 1.0000x reference)
import functools

import jax
import jax.numpy as jnp
from jax import lax
from jax.experimental import pallas as pl
from jax.experimental.pallas import tpu as pltpu
from jax.experimental.pallas import tpu_sc as plsc

N = 100000
E = 1600000
F = 32
G = 64
C_OUT = 8

NC = 2
NS = 16
LANES = 128
HALF = N // NC
HALF_PAD = 51200
CHUNK_ROWS = 8
EDGES_PER_TILE = 100352
E_PAD = EDGES_PER_TILE * NS
TILE_IDX_ROWS = EDGES_PER_TILE // LANES
N_CHUNKS = EDGES_PER_TILE // (CHUNK_ROWS * LANES)
ZROWS = 128
OUT_ROWS_PER_TILE = HALF // NS

_MESH = plsc.VectorSubcoreMesh(core_axis_name="c", subcore_axis_name="s")


@functools.partial(
    pl.kernel,
    mesh=_MESH,
    out_type=jax.ShapeDtypeStruct((N, F), jnp.float32),
    scratch_types=[
        pltpu.VMEM((CHUNK_ROWS, LANES), jnp.int32),
        pltpu.VMEM((CHUNK_ROWS, LANES), jnp.int32),
        pltpu.VMEM((CHUNK_ROWS, LANES, F), jnp.float32),
        pltpu.VMEM((ZROWS, F), jnp.float32),
        pltpu.VMEM_SHARED((HALF_PAD, F), jnp.float32),
        pltpu.SemaphoreType.DMA,
    ],
)
def _sc_agg(src_hbm, dst_hbm, x_hbm, out_hbm, src_c, dst_c, rows, zeros_v,
            agg_sh, sem):
    c = lax.axis_index("c")
    s = lax.axis_index("s")
    base = c * HALF

    def zfill(i, carry):
        zeros_v[i // 2, pl.ds((i % 2) * 16, 16)] = jnp.zeros((16,), jnp.float32)
        return carry
    lax.fori_loop(0, ZROWS * 2, zfill, 0)

    zbase = s * (HALF_PAD // NS)
    def zcopy(i, carry):
        pltpu.sync_copy(zeros_v, agg_sh.at[pl.ds(zbase + i * ZROWS, ZROWS)])
        return carry
    lax.fori_loop(0, HALF_PAD // NS // ZROWS, zcopy, 0)
    plsc.subcore_barrier()

    row0 = s * TILE_IDX_ROWS
    def chunk_body(k, carry):
        r = row0 + k * CHUNK_ROWS
        pltpu.sync_copy(src_hbm.at[pl.ds(r, CHUNK_ROWS)], src_c)
        pltpu.sync_copy(dst_hbm.at[pl.ds(r, CHUNK_ROWS)], dst_c)
        for j in range(CHUNK_ROWS):
            for t in range(LANES // 16):
                sl = pl.ds(t * 16, 16)
                lv = dst_c[j, sl] - base
                ok = (lv >= 0) & (lv < HALF)
                dst_c[j, sl] = jnp.where(ok, lv, HALF)
        cps = [pltpu.async_copy(x_hbm.at[src_c.at[j]], rows.at[j], sem)
               for j in range(CHUNK_ROWS)]
        for cp in cps:
            cp.wait()
        for j in range(CHUNK_ROWS):
            pltpu.sync_copy(rows.at[j], agg_sh.at[dst_c.at[j]], add=True)
        return carry
    lax.fori_loop(0, N_CHUNKS, chunk_body, 0)
    plsc.subcore_barrier()

    ob = s * OUT_ROWS_PER_TILE
    pltpu.sync_copy(agg_sh.at[pl.ds(ob, OUT_ROWS_PER_TILE)],
                    out_hbm.at[pl.ds(base + ob, OUT_ROWS_PER_TILE)])


_BLK = 1000
_NBLK = N // _BLK


def _layer_body(agg_ref, x_ref, wrel_ref, brel_ref, wroot_ref, out_ref, *,
                lrelu):
    h = (jnp.dot(agg_ref[...], wrel_ref[...],
                 preferred_element_type=jnp.float32)
         + brel_ref[...]
         + jnp.dot(x_ref[...], wroot_ref[...],
                   preferred_element_type=jnp.float32))
    if lrelu:
        h = jnp.where(h >= 0, h, 0.01 * h)
    out_ref[...] = h


def _tc_layer(agg, x, wrelT, brel2d, wrootT, lrelu):
    return pl.pallas_call(
        functools.partial(_layer_body, lrelu=lrelu),
        grid=(_NBLK,),
        in_specs=[
            pl.BlockSpec((_BLK, F), lambda i: (i, 0)),
            pl.BlockSpec((_BLK, F), lambda i: (i, 0)),
            pl.BlockSpec((F, F), lambda i: (0, 0)),
            pl.BlockSpec((1, F), lambda i: (0, 0)),
            pl.BlockSpec((F, F), lambda i: (0, 0)),
        ],
        out_specs=pl.BlockSpec((_BLK, F), lambda i: (i, 0)),
        out_shape=jax.ShapeDtypeStruct((N, F), jnp.float32),
    )(agg, x, wrelT, brel2d, wrootT)


def _pool_body(h_ref, b_ref, wlin_ref, blin_ref, out_ref, sums_ref, cnts_ref):
    i = pl.program_id(0)
    b = b_ref[0, 0, :]
    gids = lax.broadcasted_iota(jnp.int32, (G, _BLK), 0)
    onehot = (b[None, :] == gids).astype(jnp.float32)
    part = jnp.dot(onehot, h_ref[...], preferred_element_type=jnp.float32)
    cnt = jnp.sum(onehot, axis=1, keepdims=True)

    @pl.when(i == 0)
    def _():
        sums_ref[...] = part
        cnts_ref[...] = cnt

    @pl.when(i > 0)
    def _():
        sums_ref[...] += part
        cnts_ref[...] += cnt

    @pl.when(i == _NBLK - 1)
    def _():
        pooled = sums_ref[...] / jnp.maximum(cnts_ref[...], 1.0)
        out_ref[...] = (jnp.dot(pooled, wlin_ref[...],
                                preferred_element_type=jnp.float32)
                        + blin_ref[...])


def _tc_pool(h, batch3d, wlinT, blin2d):
    return pl.pallas_call(
        _pool_body,
        grid=(_NBLK,),
        in_specs=[
            pl.BlockSpec((_BLK, F), lambda i: (i, 0)),
            pl.BlockSpec((1, 1, _BLK), lambda i: (i, 0, 0)),
            pl.BlockSpec((F, C_OUT), lambda i: (0, 0)),
            pl.BlockSpec((1, C_OUT), lambda i: (0, 0)),
        ],
        out_specs=pl.BlockSpec((G, C_OUT), lambda i: (0, 0)),
        out_shape=jax.ShapeDtypeStruct((G, C_OUT), jnp.float32),
        scratch_shapes=[
            pltpu.VMEM((G, F), jnp.float32),
            pltpu.VMEM((G, 1), jnp.float32),
        ],
    )(h, batch3d, wlinT, blin2d)


def kernel(x, edge_index, batch, Wrel1, brel1, Wroot1, Wrel2, brel2, Wroot2,
           Wrel3, brel3, Wroot3, Wlin, blin):
    pad = E_PAD - E
    src2d = jnp.concatenate(
        [edge_index[0], jnp.zeros((pad,), jnp.int32)]).reshape(-1, LANES)
    dst2d = jnp.concatenate(
        [edge_index[1], jnp.full((pad,), -(1 << 20), jnp.int32)]
    ).reshape(-1, LANES)
    batch3d = batch.reshape(_NBLK, 1, _BLK)

    h = x
    for Wrel, brel, Wroot, lr in (
            (Wrel1, brel1, Wroot1, True),
            (Wrel2, brel2, Wroot2, True),
            (Wrel3, brel3, Wroot3, False)):
        agg = _sc_agg(src2d, dst2d, h)
        h = _tc_layer(agg, h, Wrel.T, brel.reshape(1, F), Wroot.T, lr)
    return _tc_pool(h, batch3d, Wlin.T, blin.reshape(1, C_OUT))


# speedup vs baseline: 6.3811x; 6.3811x over previous
import functools

import jax
import jax.numpy as jnp
from jax import lax
from jax.experimental import pallas as pl
from jax.experimental.pallas import tpu as pltpu
from jax.experimental.pallas import tpu_sc as plsc

N = 100000
E = 1600000
F = 32
G = 64
C_OUT = 8

NC = 2
NS = 16
LANES = 128
HALF = N // NC
HALF_PAD = 51200
CHUNK_ROWS = 4
EDGES_PER_TILE = 100352
E_PAD = EDGES_PER_TILE * NS
TILE_IDX_ROWS = EDGES_PER_TILE // LANES
N_CHUNKS = EDGES_PER_TILE // (CHUNK_ROWS * LANES)
ZROWS = 128
OUT_ROWS_PER_TILE = 3128

_MESH = plsc.VectorSubcoreMesh(core_axis_name="c", subcore_axis_name="s")


@functools.partial(
    pl.kernel,
    mesh=_MESH,
    compiler_params=pltpu.CompilerParams(use_tc_tiling_on_sc=False),
    out_type=jax.ShapeDtypeStruct((N, F), jnp.float32),
    scratch_types=[
        pltpu.VMEM((CHUNK_ROWS, LANES), jnp.int32),
        pltpu.VMEM((CHUNK_ROWS, LANES), jnp.int32),
        pltpu.VMEM((CHUNK_ROWS, LANES, F), jnp.float32),
        pltpu.VMEM_SHARED((HALF_PAD, F), jnp.float32),
        pltpu.SemaphoreType.DMA,
    ],
)
def _sc_agg(src_hbm, dst_hbm, x_hbm, out_hbm, src_c, dst_c, rows,
            agg_sh, sem):
    c = lax.axis_index("c")
    s = lax.axis_index("s")
    base = c * HALF

    def zfill(i, carry):
        rows[0, i // 2, pl.ds((i % 2) * 16, 16)] = jnp.zeros((16,), jnp.float32)
        return carry
    lax.fori_loop(0, ZROWS * 2, zfill, 0)

    zbase = s * (HALF_PAD // NS)
    def zcopy(i, carry):
        zoff = pl.multiple_of(zbase + i * ZROWS, ZROWS)
        pltpu.sync_copy(rows.at[0], agg_sh.at[pl.ds(zoff, ZROWS)])
        return carry
    lax.fori_loop(0, HALF_PAD // NS // ZROWS, zcopy, 0)
    plsc.subcore_barrier()

    row0 = s * TILE_IDX_ROWS
    def chunk_body(k, carry):
        r = pl.multiple_of(row0 + k * CHUNK_ROWS, CHUNK_ROWS)
        pltpu.sync_copy(src_hbm.at[pl.ds(r, CHUNK_ROWS)], src_c)
        pltpu.sync_copy(dst_hbm.at[pl.ds(r, CHUNK_ROWS)], dst_c)
        for j in range(CHUNK_ROWS):
            for t in range(LANES // 16):
                sl = pl.ds(t * 16, 16)
                lv = dst_c[j, sl] - base
                ok = (lv >= 0) & (lv < HALF)
                dst_c[j, sl] = jnp.where(ok, lv, HALF)
        cps = [pltpu.async_copy(x_hbm.at[src_c.at[j]], rows.at[j], sem)
               for j in range(CHUNK_ROWS)]
        for cp in cps:
            cp.wait()
        for j in range(CHUNK_ROWS):
            pltpu.sync_copy(rows.at[j], agg_sh.at[dst_c.at[j]], add=True)
        return carry
    lax.fori_loop(0, N_CHUNKS, chunk_body, 0)
    plsc.subcore_barrier()

    ob = pl.multiple_of(
        jnp.where(s == NS - 1, HALF - OUT_ROWS_PER_TILE, s * OUT_ROWS_PER_TILE),
        8)
    oo = pl.multiple_of(base + ob, 8)
    pltpu.sync_copy(agg_sh.at[pl.ds(ob, OUT_ROWS_PER_TILE)],
                    out_hbm.at[pl.ds(oo, OUT_ROWS_PER_TILE)])


_BLK = 1000
_NBLK = N // _BLK


def _layer_body(agg_ref, x_ref, wrel_ref, brel_ref, wroot_ref, out_ref, *,
                lrelu):
    h = (jnp.dot(agg_ref[...], wrel_ref[...],
                 preferred_element_type=jnp.float32)
         + brel_ref[...]
         + jnp.dot(x_ref[...], wroot_ref[...],
                   preferred_element_type=jnp.float32))
    if lrelu:
        h = jnp.where(h >= 0, h, 0.01 * h)
    out_ref[...] = h


def _tc_layer(agg, x, wrelT, brel2d, wrootT, lrelu):
    return pl.pallas_call(
        functools.partial(_layer_body, lrelu=lrelu),
        grid=(_NBLK,),
        in_specs=[
            pl.BlockSpec((_BLK, F), lambda i: (i, 0)),
            pl.BlockSpec((_BLK, F), lambda i: (i, 0)),
            pl.BlockSpec((F, F), lambda i: (0, 0)),
            pl.BlockSpec((1, F), lambda i: (0, 0)),
            pl.BlockSpec((F, F), lambda i: (0, 0)),
        ],
        out_specs=pl.BlockSpec((_BLK, F), lambda i: (i, 0)),
        out_shape=jax.ShapeDtypeStruct((N, F), jnp.float32),
    )(agg, x, wrelT, brel2d, wrootT)


def _pool_body(h_ref, b_ref, wlin_ref, blin_ref, out_ref, sums_ref, cnts_ref):
    i = pl.program_id(0)
    b = b_ref[0, 0, :]
    gids = lax.broadcasted_iota(jnp.int32, (G, _BLK), 0)
    onehot = (b[None, :] == gids).astype(jnp.float32)
    part = jnp.dot(onehot, h_ref[...], preferred_element_type=jnp.float32)
    cnt = jnp.sum(onehot, axis=1, keepdims=True)

    @pl.when(i == 0)
    def _():
        sums_ref[...] = part
        cnts_ref[...] = cnt

    @pl.when(i > 0)
    def _():
        sums_ref[...] += part
        cnts_ref[...] += cnt

    @pl.when(i == _NBLK - 1)
    def _():
        pooled = sums_ref[...] / jnp.maximum(cnts_ref[...], 1.0)
        out_ref[...] = (jnp.dot(pooled, wlin_ref[...],
                                preferred_element_type=jnp.float32)
                        + blin_ref[...])


def _tc_pool(h, batch3d, wlinT, blin2d):
    return pl.pallas_call(
        _pool_body,
        grid=(_NBLK,),
        in_specs=[
            pl.BlockSpec((_BLK, F), lambda i: (i, 0)),
            pl.BlockSpec((1, 1, _BLK), lambda i: (i, 0, 0)),
            pl.BlockSpec((F, C_OUT), lambda i: (0, 0)),
            pl.BlockSpec((1, C_OUT), lambda i: (0, 0)),
        ],
        out_specs=pl.BlockSpec((G, C_OUT), lambda i: (0, 0)),
        out_shape=jax.ShapeDtypeStruct((G, C_OUT), jnp.float32),
        scratch_shapes=[
            pltpu.VMEM((G, F), jnp.float32),
            pltpu.VMEM((G, 1), jnp.float32),
        ],
    )(h, batch3d, wlinT, blin2d)


def kernel(x, edge_index, batch, Wrel1, brel1, Wroot1, Wrel2, brel2, Wroot2,
           Wrel3, brel3, Wroot3, Wlin, blin):
    pad = E_PAD - E
    src2d = jnp.concatenate(
        [edge_index[0], jnp.zeros((pad,), jnp.int32)]).reshape(-1, LANES)
    dst2d = jnp.concatenate(
        [edge_index[1], jnp.full((pad,), -(1 << 20), jnp.int32)]
    ).reshape(-1, LANES)
    batch3d = batch.reshape(_NBLK, 1, _BLK)

    h = x
    for Wrel, brel, Wroot, lr in (
            (Wrel1, brel1, Wroot1, True),
            (Wrel2, brel2, Wroot2, True),
            (Wrel3, brel3, Wroot3, False)):
        agg = _sc_agg(src2d, dst2d, h)
        h = _tc_layer(agg, h, Wrel.T, brel.reshape(1, F), Wroot.T, lr)
    return _tc_pool(h, batch3d, Wlin.T, blin.reshape(1, C_OUT))
